# baseline (speedup 1.0000x reference)
import functools
import math

import jax
import jax.numpy as jnp
from jax import lax
from jax.experimental import pallas as pl
from jax.experimental.pallas import tpu as pltpu
from jax.experimental.pallas import tpu_sc as plsc

N = 10000
E = 320000
D = 128
EPS = 1e-5
BN_SCALE = 1.0 / math.sqrt(1.0 + EPS)

NC = 2
NS = 16
NW = NC * NS
DH = D // NC
K = 128
NBUF = 5
E_PAD = 327680
NP = 10240
PAD_ROW = 10000
RPT = NP // NS
DEG_W = 16

NCH_A = E_PAD // (NS * K)
NCH_D = NCH_A // NC
EW_ES = E_PAD // NW
NCH_ES = EW_ES // K
NG_ES = NCH_ES // 4

BR = 1000


@functools.cache
def _sc_mesh():
    return plsc.VectorSubcoreMesh(core_axis_name="c", subcore_axis_name="s",
                                  num_cores=NC, num_subcores=NS)


@functools.cache
def _deg_kernel():
    @functools.partial(
        pl.kernel,
        out_type=jax.ShapeDtypeStruct((NC, NP, DEG_W), jnp.float32),
        mesh=_sc_mesh(),
        compiler_params=pltpu.CompilerParams(use_tc_tiling_on_sc=False),
        scratch_types=[
            pltpu.VMEM_SHARED((NP, DEG_W), jnp.float32),
            pltpu.VMEM((NCH_D, K), jnp.int32),
            pltpu.VMEM((K, DEG_W), jnp.float32),
        ],
    )
    def deg(dst3_hbm, ones_hbm, zeros_hbm, out_hbm, acc_sh, didx, ones_v):
        c = lax.axis_index("c")
        s = lax.axis_index("s")
        pltpu.sync_copy(zeros_hbm, acc_sh.at[pl.ds(s * RPT, RPT)])
        pltpu.sync_copy(dst3_hbm.at[s, pl.ds(c * NCH_D, NCH_D)], didx)
        pltpu.sync_copy(ones_hbm, ones_v)
        plsc.subcore_barrier()

        def body(i, carry):
            pltpu.sync_copy(ones_v, acc_sh.at[didx.at[i]], add=True)
            return carry

        lax.fori_loop(0, NCH_D, body, 0)
        plsc.subcore_barrier()
        pltpu.sync_copy(acc_sh.at[pl.ds(s * RPT, RPT)],
                        out_hbm.at[c, pl.ds(s * RPT, RPT)])

    return deg


@functools.cache
def _agg_kernel():
    @functools.partial(
        pl.kernel,
        out_type=jax.ShapeDtypeStruct((NC, NP, D), jnp.float32),
        mesh=_sc_mesh(),
        compiler_params=pltpu.CompilerParams(use_tc_tiling_on_sc=False),
        scratch_types=[
            pltpu.VMEM_SHARED((NP, D), jnp.float32),
        ] + [pltpu.VMEM((K,), jnp.int32)] * 8
          + [pltpu.VMEM((K, D), jnp.float32)] * 2
          + [pltpu.SemaphoreType.DMA] * 10,
    )
    def agg(s_hbm, src_hbm, dst_hbm, zeros_hbm, out_hbm, acc_sh, *rest):
        sidx = rest[0:4]
        didx = rest[4:8]
        bufs = rest[8:10]
        isems = rest[10:14]
        gsems = rest[14:16]
        ssems = rest[16:20]
        c = lax.axis_index("c")
        s = lax.axis_index("s")
        base0 = (c * NS + s) * EW_ES
        pltpu.sync_copy(zeros_hbm, acc_sh.at[pl.ds(s * RPT, RPT)])
        plsc.subcore_barrier()

        def start_idx(ch, slot):
            b = base0 + ch * K
            pltpu.async_copy(src_hbm.at[pl.ds(b, K)], sidx[slot], isems[slot])
            pltpu.async_copy(dst_hbm.at[pl.ds(b, K)], didx[slot], isems[slot])

        def wait_idx(ch, slot):
            b = base0 + ch * K
            pltpu.make_async_copy(src_hbm.at[pl.ds(b, K)], sidx[slot],
                                  isems[slot]).wait()
            pltpu.make_async_copy(dst_hbm.at[pl.ds(b, K)], didx[slot],
                                  isems[slot]).wait()

        for j in range(2):
            start_idx(j, j)

        def group(i, carry):
            for j in range(4):
                ch = 4 * i + j
                j2 = (j + 2) % 4

                @pl.when(ch >= 2)
                def _():
                    pltpu.make_async_copy(bufs[j % 2], acc_sh.at[didx[j2]],
                                          ssems[j2]).wait()

                @pl.when(ch + 2 < NCH_ES)
                def _():
                    start_idx(ch + 2, j2)

                wait_idx(ch, j)
                pltpu.async_copy(s_hbm.at[sidx[j]], bufs[j % 2], gsems[j % 2])
                pltpu.make_async_copy(s_hbm.at[sidx[j]], bufs[j % 2],
                                      gsems[j % 2]).wait()
                pltpu.async_copy(bufs[j % 2], acc_sh.at[didx[j]], ssems[j],
                                 add=True)
            return carry

        lax.fori_loop(0, NG_ES, group, 0)
        for ch in (NCH_ES - 2, NCH_ES - 1):
            j = ch % 4
            pltpu.make_async_copy(bufs[j % 2], acc_sh.at[didx[j]],
                                  ssems[j]).wait()
        plsc.subcore_barrier()
        pltpu.sync_copy(acc_sh.at[pl.ds(s * RPT, RPT)],
                        out_hbm.at[c, pl.ds(s * RPT, RPT)])

    return agg


def _dis_from(deg_ref):
    deg = 1.0 + deg_ref[0, :, 0] + deg_ref[1, :, 0]
    return lax.rsqrt(deg)[:, None]


def _tc_mm_body(x_ref, w_ref, out_ref):
    out_ref[...] = jnp.dot(x_ref[...], w_ref[...],
                           preferred_element_type=jnp.float32)


def _tc_scale_body(hw_ref, deg_ref, out_ref):
    out_ref[...] = hw_ref[...] * _dis_from(deg_ref)


def _tc_mid_body(acc_ref, sprev_ref, deg_ref, b_ref, g_ref, be_ref, w_ref,
                 h_ref, snext_ref):
    dis = _dis_from(deg_ref)
    agg = (acc_ref[0] + acc_ref[1] + sprev_ref[...]) * dis
    h = jnp.maximum((agg + b_ref[0]) * (g_ref[0] * BN_SCALE) + be_ref[0], 0.0)
    h_ref[...] = h
    snext_ref[...] = jnp.dot(h, w_ref[...],
                             preferred_element_type=jnp.float32) * dis


def _tc_head_body(h_ref, wl_ref, bl_ref, out_ref):
    out_ref[...] = (
        jnp.dot(h_ref[0], wl_ref[0], preferred_element_type=jnp.float32)
        + jnp.dot(h_ref[1], wl_ref[1], preferred_element_type=jnp.float32)
        + jnp.dot(h_ref[2], wl_ref[2], preferred_element_type=jnp.float32)
        + bl_ref[0]
    )


_row_spec = pl.BlockSpec((BR, D), lambda i: (i, 0))
_acc_spec = pl.BlockSpec((NC, BR, D), lambda i: (0, i, 0))
_deg_spec = pl.BlockSpec((NC, BR, DEG_W), lambda i: (0, i, 0))
_w_spec = pl.BlockSpec((D, D), lambda i: (0, 0))
_vec_spec = pl.BlockSpec((1, D), lambda i: (0, 0))

_s_shape = jax.ShapeDtypeStruct((N, D), jnp.float32)

_tc_mm = pl.pallas_call(
    _tc_mm_body,
    grid=(N // BR,),
    in_specs=[_row_spec, _w_spec],
    out_specs=_row_spec,
    out_shape=_s_shape,
)

_tc_scale = pl.pallas_call(
    _tc_scale_body,
    grid=(N // BR,),
    in_specs=[_row_spec, _deg_spec],
    out_specs=_row_spec,
    out_shape=_s_shape,
)

_tc_mid = pl.pallas_call(
    _tc_mid_body,
    grid=(N // BR,),
    in_specs=[_acc_spec, _row_spec, _deg_spec,
              _vec_spec, _vec_spec, _vec_spec, _w_spec],
    out_specs=(_row_spec, _row_spec),
    out_shape=(jax.ShapeDtypeStruct((N, D), jnp.float32), _s_shape),
)

_tc_head = pl.pallas_call(
    _tc_head_body,
    grid=(N // BR,),
    in_specs=[pl.BlockSpec((3, BR, D), lambda i: (0, i, 0)),
              pl.BlockSpec((3, D, D), lambda i: (0, 0, 0)), _vec_spec],
    out_specs=_row_spec,
    out_shape=jax.ShapeDtypeStruct((N, D), jnp.float32),
)


def kernel(x, edge_index, W0, b0, g0, be0, W1, b1, g1, be1, W2, b2, g2, be2,
           Wl, bl):
    zeros_rows = jnp.zeros((RPT, D), jnp.float32)
    zeros_deg = jnp.zeros((RPT, DEG_W), jnp.float32)
    ones_deg = jnp.ones((K, DEG_W), jnp.float32)
    r2 = lambda v: v.reshape(1, D)

    deg_k = _deg_kernel()
    agg_k = _agg_kernel()
    npad = E_PAD - E
    pad_src = (jnp.arange(npad, dtype=jnp.int32) * 37) % N
    pad_dst = PAD_ROW + (jnp.arange(npad, dtype=jnp.int32) % (NP - PAD_ROW))
    src_ids = jnp.concatenate([edge_index[0], pad_src])
    dst_ids = jnp.concatenate([edge_index[1], pad_dst])
    dst3 = dst_ids.reshape(NS, NCH_A, K)

    degout = deg_k(dst3, ones_deg, zeros_deg)
    hw0 = _tc_mm(x, W0)
    s0 = _tc_scale(hw0, degout)

    Ws = jnp.stack([W1, W2, jnp.zeros((D, D), jnp.float32)])
    bs = jnp.stack([r2(b0), r2(b1), r2(b2)])
    gs = jnp.stack([r2(g0), r2(g1), r2(g2)])
    bes = jnp.stack([r2(be0), r2(be1), r2(be2)])

    def layer(s_in, params):
        W_next, b, g, be = params
        acc = agg_k(s_in, src_ids, dst_ids, zeros_rows)
        h, s_next = _tc_mid(acc, s_in, degout, b, g, be, W_next)
        return s_next, h

    _, hstack = jax.lax.scan(layer, s0, (Ws, bs, gs, bes))
    return _tc_head(hstack, Wl.reshape(3, D, D), bl.reshape(1, D))


# speedup vs baseline: 1.0066x; 1.0066x over previous
import functools
import math

import jax
import jax.numpy as jnp
from jax import lax
from jax.experimental import pallas as pl
from jax.experimental.pallas import tpu as pltpu
from jax.experimental.pallas import tpu_sc as plsc

N = 10000
E = 320000
D = 128
EPS = 1e-5
BN_SCALE = 1.0 / math.sqrt(1.0 + EPS)

NC = 2
NS = 16
NW = NC * NS
DH = D // NC
K = 128
NBUF = 5
E_PAD = 327680
NP = 10240
PAD_ROW = 10000
RPT = NP // NS
DEG_W = 16

NCH_A = E_PAD // (NS * K)
NCH_D = NCH_A // NC
EW_ES = E_PAD // NW
NCH_ES = EW_ES // K
NG_ES = NCH_ES // 4

BR = 1000


@functools.cache
def _sc_mesh():
    return plsc.VectorSubcoreMesh(core_axis_name="c", subcore_axis_name="s",
                                  num_cores=NC, num_subcores=NS)


@functools.cache
def _deg_kernel():
    @functools.partial(
        pl.kernel,
        out_type=jax.ShapeDtypeStruct((NC, NP, DEG_W), jnp.float32),
        mesh=_sc_mesh(),
        compiler_params=pltpu.CompilerParams(use_tc_tiling_on_sc=False),
        scratch_types=[
            pltpu.VMEM_SHARED((NP, DEG_W), jnp.float32),
            pltpu.VMEM((NCH_D, K), jnp.int32),
            pltpu.VMEM((K, DEG_W), jnp.float32),
        ],
    )
    def deg(dst3_hbm, ones_hbm, zeros_hbm, out_hbm, acc_sh, didx, ones_v):
        c = lax.axis_index("c")
        s = lax.axis_index("s")
        pltpu.sync_copy(zeros_hbm, acc_sh.at[pl.ds(s * RPT, RPT)])
        pltpu.sync_copy(dst3_hbm.at[s, pl.ds(c * NCH_D, NCH_D)], didx)
        pltpu.sync_copy(ones_hbm, ones_v)
        plsc.subcore_barrier()

        def body(i, carry):
            pltpu.sync_copy(ones_v, acc_sh.at[didx.at[i]], add=True)
            return carry

        lax.fori_loop(0, NCH_D, body, 0)
        plsc.subcore_barrier()
        pltpu.sync_copy(acc_sh.at[pl.ds(s * RPT, RPT)],
                        out_hbm.at[c, pl.ds(s * RPT, RPT)])

    return deg


@functools.cache
def _agg_kernel():
    @functools.partial(
        pl.kernel,
        out_type=jax.ShapeDtypeStruct((NC, NP, D), jnp.float32),
        mesh=_sc_mesh(),
        compiler_params=pltpu.CompilerParams(use_tc_tiling_on_sc=False),
        scratch_types=[
            pltpu.VMEM_SHARED((NP, D), jnp.float32),
        ] + [pltpu.VMEM((K,), jnp.int32)] * 8
          + [pltpu.VMEM((K, D), jnp.float32)] * 2
          + [pltpu.SemaphoreType.DMA] * 10,
    )
    def agg(s_hbm, src_hbm, dst_hbm, zeros_hbm, out_hbm, acc_sh, *rest):
        sidx = rest[0:4]
        didx = rest[4:8]
        bufs = rest[8:10]
        isems = rest[10:14]
        gsems = rest[14:16]
        ssems = rest[16:20]
        c = lax.axis_index("c")
        s = lax.axis_index("s")
        base0 = (c * NS + s) * EW_ES
        pltpu.sync_copy(zeros_hbm, acc_sh.at[pl.ds(s * RPT, RPT)])
        plsc.subcore_barrier()

        def start_idx(ch, slot):
            b = base0 + ch * K
            pltpu.async_copy(src_hbm.at[pl.ds(b, K)], sidx[slot], isems[slot])
            pltpu.async_copy(dst_hbm.at[pl.ds(b, K)], didx[slot], isems[slot])

        def wait_idx(ch, slot):
            b = base0 + ch * K
            pltpu.make_async_copy(src_hbm.at[pl.ds(b, K)], sidx[slot],
                                  isems[slot]).wait()
            pltpu.make_async_copy(dst_hbm.at[pl.ds(b, K)], didx[slot],
                                  isems[slot]).wait()

        for j in range(2):
            start_idx(j, j)

        def group(i, carry):
            for j in range(4):
                ch = 4 * i + j
                j2 = (j + 2) % 4

                @pl.when(ch >= 2)
                def _():
                    pltpu.make_async_copy(bufs[j % 2], acc_sh.at[didx[j2]],
                                          ssems[j2]).wait()

                @pl.when(ch + 2 < NCH_ES)
                def _():
                    start_idx(ch + 2, j2)

                wait_idx(ch, j)
                pltpu.async_copy(s_hbm.at[sidx[j]], bufs[j % 2], gsems[j % 2])
                pltpu.make_async_copy(s_hbm.at[sidx[j]], bufs[j % 2],
                                      gsems[j % 2]).wait()
                pltpu.async_copy(bufs[j % 2], acc_sh.at[didx[j]], ssems[j],
                                 add=True)
            return carry

        lax.fori_loop(0, NG_ES, group, 0)
        for ch in (NCH_ES - 2, NCH_ES - 1):
            j = ch % 4
            pltpu.make_async_copy(bufs[j % 2], acc_sh.at[didx[j]],
                                  ssems[j]).wait()
        plsc.subcore_barrier()
        pltpu.sync_copy(acc_sh.at[pl.ds(s * RPT, RPT)],
                        out_hbm.at[c, pl.ds(s * RPT, RPT)])

    return agg


def _dis_from(deg_ref):
    deg = 1.0 + deg_ref[0, :, 0] + deg_ref[1, :, 0]
    return lax.rsqrt(deg)[:, None]


def _tc_first_body(x_ref, w_ref, deg_ref, out_ref):
    hw = jnp.dot(x_ref[...], w_ref[...], preferred_element_type=jnp.float32)
    out_ref[...] = hw * _dis_from(deg_ref)


def _tc_mid_body(acc_ref, sprev_ref, deg_ref, b_ref, g_ref, be_ref, w_ref,
                 h_ref, snext_ref):
    dis = _dis_from(deg_ref)
    agg = (acc_ref[0] + acc_ref[1] + sprev_ref[...]) * dis
    h = jnp.maximum((agg + b_ref[0]) * (g_ref[0] * BN_SCALE) + be_ref[0], 0.0)
    h_ref[...] = h
    snext_ref[...] = jnp.dot(h, w_ref[...],
                             preferred_element_type=jnp.float32) * dis


def _tc_head_body(h_ref, wl_ref, bl_ref, out_ref):
    out_ref[...] = (
        jnp.dot(h_ref[0], wl_ref[0], preferred_element_type=jnp.float32)
        + jnp.dot(h_ref[1], wl_ref[1], preferred_element_type=jnp.float32)
        + jnp.dot(h_ref[2], wl_ref[2], preferred_element_type=jnp.float32)
        + bl_ref[0]
    )


_row_spec = pl.BlockSpec((BR, D), lambda i: (i, 0))
_acc_spec = pl.BlockSpec((NC, BR, D), lambda i: (0, i, 0))
_deg_spec = pl.BlockSpec((NC, BR, DEG_W), lambda i: (0, i, 0))
_w_spec = pl.BlockSpec((D, D), lambda i: (0, 0))
_vec_spec = pl.BlockSpec((1, D), lambda i: (0, 0))

_s_shape = jax.ShapeDtypeStruct((N, D), jnp.float32)

_tc_first = pl.pallas_call(
    _tc_first_body,
    grid=(N // BR,),
    in_specs=[_row_spec, _w_spec, _deg_spec],
    out_specs=_row_spec,
    out_shape=_s_shape,
)

_tc_mid = pl.pallas_call(
    _tc_mid_body,
    grid=(N // BR,),
    in_specs=[_acc_spec, _row_spec, _deg_spec,
              _vec_spec, _vec_spec, _vec_spec, _w_spec],
    out_specs=(_row_spec, _row_spec),
    out_shape=(jax.ShapeDtypeStruct((N, D), jnp.float32), _s_shape),
)

_tc_head = pl.pallas_call(
    _tc_head_body,
    grid=(N // BR,),
    in_specs=[pl.BlockSpec((3, BR, D), lambda i: (0, i, 0)),
              pl.BlockSpec((3, D, D), lambda i: (0, 0, 0)), _vec_spec],
    out_specs=_row_spec,
    out_shape=jax.ShapeDtypeStruct((N, D), jnp.float32),
)


def kernel(x, edge_index, W0, b0, g0, be0, W1, b1, g1, be1, W2, b2, g2, be2,
           Wl, bl):
    zeros_rows = jnp.zeros((RPT, D), jnp.float32)
    zeros_deg = jnp.zeros((RPT, DEG_W), jnp.float32)
    ones_deg = jnp.ones((K, DEG_W), jnp.float32)
    r2 = lambda v: v.reshape(1, D)

    deg_k = _deg_kernel()
    agg_k = _agg_kernel()
    npad = E_PAD - E
    pad_src = (jnp.arange(npad, dtype=jnp.int32) * 37) % N
    pad_dst = PAD_ROW + (jnp.arange(npad, dtype=jnp.int32) % (NP - PAD_ROW))
    src_ids = jnp.concatenate([edge_index[0], pad_src])
    dst_ids = jnp.concatenate([edge_index[1], pad_dst])
    dst3 = dst_ids.reshape(NS, NCH_A, K)

    degout = deg_k(dst3, ones_deg, zeros_deg)
    s0 = _tc_first(x, W0, degout)

    Ws = jnp.stack([W1, W2, jnp.zeros((D, D), jnp.float32)])
    bs = jnp.stack([r2(b0), r2(b1), r2(b2)])
    gs = jnp.stack([r2(g0), r2(g1), r2(g2)])
    bes = jnp.stack([r2(be0), r2(be1), r2(be2)])

    def layer(s_in, params):
        W_next, b, g, be = params
        acc = agg_k(s_in, src_ids, dst_ids, zeros_rows)
        h, s_next = _tc_mid(acc, s_in, degout, b, g, be, W_next)
        return s_next, h

    _, hstack = jax.lax.scan(layer, s0, (Ws, bs, gs, bes))
    return _tc_head(hstack, Wl.reshape(3, D, D), bl.reshape(1, D))
